# Initial kernel scaffold; baseline (speedup 1.0000x reference)
import jax
import jax.numpy as jnp
from jax.experimental import pallas as pl


def kernel(hidden_states, router_w, bias, Wg, Wu, Wd, Sg, Su, Sd):
    raise NotImplementedError("write your pallas kernel here")


# speedup vs baseline: 1.6683x; 1.6683x over previous
import functools

import jax
import jax.numpy as jnp
from jax import lax
from jax.experimental import pallas as pl
from jax.experimental.pallas import tpu as pltpu
from jax.experimental.pallas import tpu_sc as plsc

E = 16
TOPK = 2
NGROUP = 4
GSIZE = E // NGROUP
TOPK_GROUP = 2
D = 2048
FF = 1024
SCALE = 2.5
T = 2048
P = T * TOPK
TILE = 128
PROWS = P + E * TILE
NTILES = PROWS // TILE
TB = 256
NSC = 32

_NEG = -1e30


def _router_body(x_ref, rw_ref, bias_ref, i1_ref, i2_ref, w1_ref, w2_ref):
    xb = x_ref[...]
    rw = rw_ref[...]
    logits = lax.dot_general(xb, rw, (((1,), (1,)), ((), ())),
                             preferred_element_type=jnp.float32)
    scores = jax.nn.sigmoid(logits)
    s4c = scores + bias_ref[...]
    lanes = lax.broadcasted_iota(jnp.int32, (TB, E), 1)

    def top1(vals):
        m = jnp.max(vals, axis=1, keepdims=True)
        i = jnp.min(jnp.where(vals == m, lanes, E), axis=1, keepdims=True)
        return m, i

    gs = jnp.where(lanes < NGROUP, 0.0, _NEG)
    for g in range(NGROUP):
        vals = jnp.where(lanes // GSIZE == g, s4c, _NEG)
        m1, i1 = top1(vals)
        m2, _ = top1(jnp.where(lanes == i1, _NEG, vals))
        gs = jnp.where(lanes == g, m1 + m2, gs)
    _, gi1 = top1(gs)
    _, gi2 = top1(jnp.where(lanes == gi1, _NEG, gs))
    sel = (lanes // GSIZE == gi1) | (lanes // GSIZE == gi2)
    masked = jnp.where(sel, s4c, 0.0)
    m1, i1 = top1(masked)
    _, i2 = top1(jnp.where(lanes == i1, _NEG, masked))
    w1 = jnp.sum(jnp.where(lanes == i1, scores, 0.0), axis=1, keepdims=True)
    w2 = jnp.sum(jnp.where(lanes == i2, scores, 0.0), axis=1, keepdims=True)
    norm = SCALE / (w1 + w2 + 1e-20)
    i1_ref[...] = i1
    i2_ref[...] = i2
    w1_ref[...] = w1 * norm
    w2_ref[...] = w2 * norm


def _run_router(x2, router_w, bias2):
    outs = jax.ShapeDtypeStruct((T, 1), jnp.int32), jax.ShapeDtypeStruct((T, 1), jnp.int32), \
        jax.ShapeDtypeStruct((T, 1), jnp.float32), jax.ShapeDtypeStruct((T, 1), jnp.float32)
    col = pl.BlockSpec((TB, 1), lambda i: (i, 0))
    return pl.pallas_call(
        _router_body,
        grid=(T // TB,),
        in_specs=[pl.BlockSpec((TB, D), lambda i: (i, 0)),
                  pl.BlockSpec((E, D), lambda i: (0, 0)),
                  pl.BlockSpec((1, E), lambda i: (0, 0))],
        out_specs=(col, col, col, col),
        out_shape=outs,
    )(x2, router_w, bias2)


def _plan_body(e_ref, dest_ref, te_ref):
    e_col = e_ref[...]
    lanes = lax.broadcasted_iota(jnp.int32, (P, E), 1)
    oh = (e_col == lanes).astype(jnp.int32)
    counts = jnp.sum(oh, axis=0, keepdims=True)
    padded = ((counts + (TILE - 1)) >> 7) << 7
    r16 = lax.broadcasted_iota(jnp.int32, (E, E), 0)
    c16 = lax.broadcasted_iota(jnp.int32, (E, E), 1)
    lt = (r16 <= c16).astype(jnp.float32)
    seg_incl = lax.dot_general(padded.astype(jnp.float32), lt,
                               (((1,), (0,)), ((), ())),
                               preferred_element_type=jnp.float32,
                               precision=lax.Precision.HIGHEST).astype(jnp.int32)
    seg_start = seg_incl - padded
    csum = oh
    s = 1
    while s < P:
        shifted = jnp.concatenate(
            [jnp.zeros((s, E), jnp.int32), csum[: P - s, :]], axis=0)
        csum = csum + shifted
        s *= 2
    rank = jnp.sum(oh * csum, axis=1, keepdims=True)
    segsel = jnp.sum(oh * seg_start, axis=1, keepdims=True)
    dest_ref[...] = segsel + rank - 1
    ts = lax.broadcasted_iota(jnp.int32, (NTILES, E), 0) * TILE
    te = jnp.sum((ts >= seg_incl).astype(jnp.int32), axis=1, keepdims=True)
    te_ref[...] = jnp.minimum(te, E - 1)


def _run_plan(e_col):
    return pl.pallas_call(
        _plan_body,
        in_specs=[pl.BlockSpec((P, 1), lambda: (0, 0))],
        out_specs=(pl.BlockSpec((P, 1), lambda: (0, 0)),
                   pl.BlockSpec((NTILES, 1), lambda: (0, 0))),
        out_shape=(jax.ShapeDtypeStruct((P, 1), jnp.int32),
                   jax.ShapeDtypeStruct((NTILES, 1), jnp.int32)),
    )(e_col)


def _scatter_body(dest_hbm, gate_hbm, rt_hbm, rg_hbm, dest_v, gate_v, tok_v,
                  sem1, sem2):
    wid = lax.axis_index("s") * 2 + lax.axis_index("c")
    n = P // NSC
    base = wid * n
    pltpu.sync_copy(dest_hbm.at[pl.ds(base, n)], dest_v)
    pltpu.sync_copy(gate_hbm.at[pl.ds(base, n)], gate_v)
    iot = lax.iota(jnp.int32, 16)
    for c in range(n // 16):
        tok_v[pl.ds(c * 16, 16)] = (base + c * 16 + iot) >> 1
    cp1 = pltpu.make_async_copy(tok_v, rt_hbm.at[dest_v], sem1)
    cp2 = pltpu.make_async_copy(gate_v, rg_hbm.at[dest_v], sem2)
    cp1.start()
    cp2.start()
    cp1.wait()
    cp2.wait()


def _run_scatter(dest, gflat):
    n = P // NSC
    mesh = plsc.VectorSubcoreMesh(core_axis_name="c", subcore_axis_name="s")
    f = pl.kernel(
        _scatter_body,
        out_type=(jax.ShapeDtypeStruct((PROWS,), jnp.int32),
                  jax.ShapeDtypeStruct((PROWS,), jnp.float32)),
        mesh=mesh,
        scratch_types=[pltpu.VMEM((n,), jnp.int32),
                       pltpu.VMEM((n,), jnp.float32),
                       pltpu.VMEM((n,), jnp.int32),
                       pltpu.SemaphoreType.DMA,
                       pltpu.SemaphoreType.DMA],
    )
    return f(dest, gflat)


def _gather_body(x_hbm, rt_hbm, xs_hbm, raw_v, idx_v, rows_v, sem):
    wid = lax.axis_index("s") * 2 + lax.axis_index("c")
    per = PROWS // NSC
    chunk = 48
    for ci in range(per // chunk):
        o = wid * per + ci * chunk
        pltpu.sync_copy(rt_hbm.at[pl.ds(o, chunk)], raw_v)
        for c in range(chunk // 16):
            idx_v[pl.ds(c * 16, 16)] = raw_v[pl.ds(c * 16, 16)] & (T - 1)
        cp = pltpu.make_async_copy(x_hbm.at[idx_v], rows_v, sem)
        cp.start()
        cp.wait()
        pltpu.sync_copy(rows_v, xs_hbm.at[pl.ds(o, chunk)])


def _run_gather(x2, row_token):
    mesh = plsc.VectorSubcoreMesh(core_axis_name="c", subcore_axis_name="s")
    f = pl.kernel(
        _gather_body,
        out_type=jax.ShapeDtypeStruct((PROWS, D), jnp.float32),
        mesh=mesh,
        scratch_types=[pltpu.VMEM((48,), jnp.int32),
                       pltpu.VMEM((48,), jnp.int32),
                       pltpu.VMEM((48, D), jnp.float32),
                       pltpu.SemaphoreType.DMA],
    )
    return f(x2, row_token)


def _ffn_body(te_ref, xs_ref, gate_ref, wg_ref, wu_ref, wd_ref, out_ref):
    xb = xs_ref[...]
    g = lax.dot_general(xb, wg_ref[0], (((1,), (1,)), ((), ())),
                        preferred_element_type=jnp.float32)
    u = lax.dot_general(xb, wu_ref[0], (((1,), (1,)), ((), ())),
                        preferred_element_type=jnp.float32)
    h = g * jax.nn.sigmoid(g) * u
    o = lax.dot_general(h, wd_ref[0], (((1,), (1,)), ((), ())),
                        preferred_element_type=jnp.float32)
    out_ref[...] = o * gate_ref[...]


def _run_ffn(te, xs, row_gate_col, Wg, Wu, Wd):
    spec = pltpu.PrefetchScalarGridSpec(
        num_scalar_prefetch=1,
        grid=(NTILES,),
        in_specs=[
            pl.BlockSpec((TILE, D), lambda i, te: (i, 0)),
            pl.BlockSpec((TILE, 1), lambda i, te: (i, 0)),
            pl.BlockSpec((1, FF, D), lambda i, te: (te[i], 0, 0)),
            pl.BlockSpec((1, FF, D), lambda i, te: (te[i], 0, 0)),
            pl.BlockSpec((1, D, FF), lambda i, te: (te[i], 0, 0)),
        ],
        out_specs=pl.BlockSpec((TILE, D), lambda i, te: (i, 0)),
    )
    return pl.pallas_call(
        _ffn_body,
        grid_spec=spec,
        out_shape=jax.ShapeDtypeStruct((PROWS, D), jnp.float32),
    )(te, xs, row_gate_col, Wg, Wu, Wd)


def _combine_body(h_hbm, p0_hbm, p1_hbm, g_hbm, p0_v, p1_v, a_v, b_v, semA,
                  semB):
    wid = lax.axis_index("s") * 2 + lax.axis_index("c")
    per = T // NSC
    chunk = 16
    for ci in range(per // chunk):
        o = wid * per + ci * chunk
        pltpu.sync_copy(p0_hbm.at[pl.ds(o, chunk)], p0_v)
        pltpu.sync_copy(p1_hbm.at[pl.ds(o, chunk)], p1_v)
        cpA = pltpu.make_async_copy(h_hbm.at[p0_v], a_v, semA)
        cpB = pltpu.make_async_copy(h_hbm.at[p1_v], b_v, semB)
        cpA.start()
        cpB.start()
        cpA.wait()
        cpB.wait()

        def addb(i, carry):
            r = i >> 7
            c = i & 127
            plsc.addupdate(a_v.at[r, pl.ds(c * 16, 16)],
                           b_v[r, pl.ds(c * 16, 16)])
            return carry

        lax.fori_loop(0, chunk * (D // 16), addb, 0)
        pltpu.sync_copy(a_v, g_hbm.at[pl.ds(o, chunk)])


def _run_combine(h, pos0, pos1):
    mesh = plsc.VectorSubcoreMesh(core_axis_name="c", subcore_axis_name="s")
    f = pl.kernel(
        _combine_body,
        out_type=jax.ShapeDtypeStruct((T, D), jnp.float32),
        mesh=mesh,
        scratch_types=[pltpu.VMEM((16,), jnp.int32),
                       pltpu.VMEM((16,), jnp.int32),
                       pltpu.VMEM((16, D), jnp.float32),
                       pltpu.VMEM((16, D), jnp.float32),
                       pltpu.SemaphoreType.DMA,
                       pltpu.SemaphoreType.DMA],
    )
    return f(h, pos0, pos1)


def _shared_body(x_ref, gin_ref, sg_ref, su_ref, sd_ref, out_ref):
    xb = x_ref[...]
    g = lax.dot_general(xb, sg_ref[...], (((1,), (1,)), ((), ())),
                        preferred_element_type=jnp.float32)
    u = lax.dot_general(xb, su_ref[...], (((1,), (1,)), ((), ())),
                        preferred_element_type=jnp.float32)
    h = g * jax.nn.sigmoid(g) * u
    o = lax.dot_general(h, sd_ref[...], (((1,), (1,)), ((), ())),
                        preferred_element_type=jnp.float32)
    out_ref[...] = o + gin_ref[...]


def _run_shared(x2, gsum, Sg, Su, Sd):
    return pl.pallas_call(
        _shared_body,
        grid=(T // TILE,),
        in_specs=[pl.BlockSpec((TILE, D), lambda i: (i, 0)),
                  pl.BlockSpec((TILE, D), lambda i: (i, 0)),
                  pl.BlockSpec((FF, D), lambda i: (0, 0)),
                  pl.BlockSpec((FF, D), lambda i: (0, 0)),
                  pl.BlockSpec((D, FF), lambda i: (0, 0))],
        out_specs=pl.BlockSpec((TILE, D), lambda i: (i, 0)),
        out_shape=jax.ShapeDtypeStruct((T, D), jnp.float32),
    )(x2, gsum, Sg, Su, Sd)


def kernel(hidden_states, router_w, bias, Wg, Wu, Wd, Sg, Su, Sd):
    orig_shape = hidden_states.shape
    x2 = hidden_states.reshape(T, D)
    i1, i2, w1, w2 = _run_router(x2, router_w, bias.reshape(1, E))
    e_col = jnp.concatenate([i1, i2], axis=1).reshape(P, 1)
    gflat = jnp.concatenate([w1, w2], axis=1).reshape(P)
    dest_col, te_col = _run_plan(e_col)
    dest = dest_col.reshape(P)
    row_token, row_gate = _run_scatter(dest, gflat)
    xs = _run_gather(x2, row_token)
    h = _run_ffn(te_col.reshape(NTILES), xs, row_gate.reshape(PROWS, 1),
                 Wg, Wu, Wd)
    dest2 = dest_col.reshape(T, TOPK)
    gsum = _run_combine(h, dest2[:, 0], dest2[:, 1])
    out = _run_shared(x2, gsum, Sg, Su, Sd)
    return out.reshape(orig_shape)
